# baseline (speedup 1.0000x reference)
import functools

import jax
import jax.numpy as jnp
from jax import lax
from jax.experimental import pallas as pl
from jax.experimental.pallas import tpu as pltpu
from jax.experimental.pallas import tpu_sc as plsc

_RI = 32
_W = 192
_CH_IN = _RI * _W
_CH_OUT = 4 * _CH_IN
_NW = 32


def _sc_body(x_hbm, idx_hbm, out_hbm, xb, ib, ob):
    n_chunks = x_hbm.shape[0] // _CH_IN
    per_w = n_chunks // _NW
    chunks_per_plane = (_W * _W) // _CH_IN
    wid = lax.axis_index("s") * 2 + lax.axis_index("c")

    def chunk_body(t, carry):
        c = wid * per_w + t
        pltpu.sync_copy(x_hbm.at[pl.ds(c * _CH_IN, _CH_IN)], xb)
        pltpu.sync_copy(idx_hbm.at[pl.ds(c * _CH_IN, _CH_IN)], ib)

        def zbody(z, zc):
            ob[pl.ds(z * 16, 16)] = jnp.zeros((16,), jnp.float32)
            return zc

        lax.fori_loop(0, _CH_OUT // 16, zbody, 0, unroll=8)

        base = (c % chunks_per_plane) * (_RI * 4 * _W)

        def sbody(k, sc):
            xv = xb[pl.ds(k * 16, 16)]
            iv = ib[pl.ds(k * 16, 16)]
            plsc.store_scatter(ob, [iv - base], xv)
            return sc

        lax.fori_loop(0, _CH_IN // 16, sbody, 0, unroll=4)
        pltpu.sync_copy(ob, out_hbm.at[pl.ds(c * _CH_OUT, _CH_OUT)])
        return carry

    lax.fori_loop(0, per_w, chunk_body, 0)


def kernel(x, indices):
    b, ch, h, w = x.shape
    xf = x.reshape(-1)
    idxf = indices.astype(jnp.int32).reshape(-1)
    total_out = b * ch * 4 * h * w
    mesh = plsc.VectorSubcoreMesh(core_axis_name="c", subcore_axis_name="s")
    sc_call = functools.partial(
        pl.kernel,
        mesh=mesh,
        out_type=jax.ShapeDtypeStruct((total_out,), jnp.float32),
        scratch_types=[
            pltpu.VMEM((_CH_IN,), jnp.float32),
            pltpu.VMEM((_CH_IN,), jnp.int32),
            pltpu.VMEM((_CH_OUT,), jnp.float32),
        ],
        compiler_params=pltpu.CompilerParams(needs_layout_passes=False),
    )(_sc_body)
    out = sc_call(xf, idxf)
    return out.reshape(b, ch, 2 * h, 2 * w)


# speedup vs baseline: 1.3056x; 1.3056x over previous
import functools

import jax
import jax.numpy as jnp
from jax import lax
from jax.experimental import pallas as pl
from jax.experimental.pallas import tpu as pltpu
from jax.experimental.pallas import tpu_sc as plsc

_W = 192
_RI = 48
_CH_IN = _RI * _W
_CH_OUT = 4 * _CH_IN
_NW = 32


def _sc_body(x_hbm, idx_hbm, out_hbm,
             xb0, xb1, ib0, ib1, ob0, ob1,
             sx0, sx1, si0, si1, so0, so1):
    n_chunks = x_hbm.shape[0] // _CH_IN
    per_w = n_chunks // _NW
    cpp = (_W * _W) // _CH_IN
    wid = lax.axis_index("s") * 2 + lax.axis_index("c")
    base = wid * per_w
    xbs, ibs, obs = (xb0, xb1), (ib0, ib1), (ob0, ob1)
    sxs, sis, sos = (sx0, sx1), (si0, si1), (so0, so1)

    def in_copies(c, b):
        return (
            pltpu.make_async_copy(
                x_hbm.at[pl.ds(c * _CH_IN, _CH_IN)], xbs[b], sxs[b]),
            pltpu.make_async_copy(
                idx_hbm.at[pl.ds(c * _CH_IN, _CH_IN)], ibs[b], sis[b]),
        )

    def out_copy(c, b):
        return pltpu.make_async_copy(
            obs[b], out_hbm.at[pl.ds(c * _CH_OUT, _CH_OUT)], sos[b])

    for cc in in_copies(base, 0):
        cc.start()
    for cc in in_copies(base + 1, 1):
        cc.start()

    def pair_body(p, carry):
        for b in range(2):
            t = 2 * p + b
            c = base + t
            ob, xb, ib = obs[b], xbs[b], ibs[b]

            @pl.when(t >= 2)
            def _():
                out_copy(c - 2, b).wait()

            def zbody(z, zc):
                ob[pl.ds(z * 16, 16)] = jnp.zeros((16,), jnp.float32)
                return zc

            lax.fori_loop(0, _CH_OUT // 16, zbody, 0, unroll=8)

            for cc in in_copies(c, b):
                cc.wait()

            pbase = (c % cpp) * _CH_OUT

            def sbody(k, sc):
                xv = xb[pl.ds(k * 16, 16)]
                iv = ib[pl.ds(k * 16, 16)]
                plsc.store_scatter(ob, [iv - pbase], xv)
                return sc

            lax.fori_loop(0, _CH_IN // 16, sbody, 0, unroll=4)

            out_copy(c, b).start()

            @pl.when(t + 2 < per_w)
            def _():
                for cc in in_copies(c + 2, b):
                    cc.start()
        return carry

    lax.fori_loop(0, per_w // 2, pair_body, 0)
    out_copy(base + per_w - 2, 0).wait()
    out_copy(base + per_w - 1, 1).wait()


def kernel(x, indices):
    b, ch, h, w = x.shape
    xf = x.reshape(-1)
    idxf = indices.astype(jnp.int32).reshape(-1)
    total_out = b * ch * 4 * h * w
    mesh = plsc.VectorSubcoreMesh(core_axis_name="c", subcore_axis_name="s")
    sc_call = functools.partial(
        pl.kernel,
        mesh=mesh,
        out_type=jax.ShapeDtypeStruct((total_out,), jnp.float32),
        scratch_types=[
            pltpu.VMEM((_CH_IN,), jnp.float32),
            pltpu.VMEM((_CH_IN,), jnp.float32),
            pltpu.VMEM((_CH_IN,), jnp.int32),
            pltpu.VMEM((_CH_IN,), jnp.int32),
            pltpu.VMEM((_CH_OUT,), jnp.float32),
            pltpu.VMEM((_CH_OUT,), jnp.float32),
            pltpu.SemaphoreType.DMA,
            pltpu.SemaphoreType.DMA,
            pltpu.SemaphoreType.DMA,
            pltpu.SemaphoreType.DMA,
            pltpu.SemaphoreType.DMA,
            pltpu.SemaphoreType.DMA,
        ],
        compiler_params=pltpu.CompilerParams(needs_layout_passes=False),
    )(_sc_body)
    out = sc_call(xf, idxf)
    return out.reshape(b, ch, 2 * h, 2 * w)


# speedup vs baseline: 1.6814x; 1.2878x over previous
import functools

import jax
import jax.numpy as jnp
from jax import lax
from jax.experimental import pallas as pl
from jax.experimental.pallas import tpu as pltpu
from jax.experimental.pallas import tpu_sc as plsc

_W = 192
_RI = 48
_CH_IN = _RI * _W
_CH_OUT = 4 * _CH_IN
_NW = 32


def _sc_body(x_hbm, idx_hbm, out_hbm,
             xb0, xb1, ib0, ib1, ob0, ob1,
             sx0, sx1, si0, si1, so0, so1):
    n_chunks = x_hbm.shape[0] // _CH_IN
    per_w = n_chunks // _NW
    cpp = (_W * _W) // _CH_IN
    wid = lax.axis_index("s") * 2 + lax.axis_index("c")
    base = wid * per_w
    xbs, ibs, obs = (xb0, xb1), (ib0, ib1), (ob0, ob1)
    sxs, sis, sos = (sx0, sx1), (si0, si1), (so0, so1)

    def in_copies(c, b):
        return (
            pltpu.make_async_copy(
                x_hbm.at[pl.ds(c * _CH_IN, _CH_IN)], xbs[b], sxs[b]),
            pltpu.make_async_copy(
                idx_hbm.at[pl.ds(c * _CH_IN, _CH_IN)], ibs[b], sis[b]),
        )

    def out_copy(c, b):
        return pltpu.make_async_copy(
            obs[b], out_hbm.at[pl.ds(c * _CH_OUT, _CH_OUT)], sos[b])

    for cc in in_copies(base, 0):
        cc.start()
    for cc in in_copies(base + 1, 1):
        cc.start()

    def pair_body(p, carry):
        for b in range(2):
            t = 2 * p + b
            c = base + t
            ob, xb, ib = obs[b], xbs[b], ibs[b]

            @pl.when(t >= 2)
            def _():
                out_copy(c - 2, b).wait()

            @plsc.parallel_loop(0, _CH_OUT, step=16, unroll=8)
            def _(z):
                ob[pl.ds(z, 16)] = jnp.zeros((16,), jnp.float32)

            for cc in in_copies(c, b):
                cc.wait()

            pbase = (c % cpp) * _CH_OUT

            @plsc.parallel_loop(0, _CH_IN, step=16, unroll=4)
            def _(k):
                xv = xb[pl.ds(k, 16)]
                iv = ib[pl.ds(k, 16)]
                plsc.store_scatter(ob, [iv - pbase], xv)

            out_copy(c, b).start()

            @pl.when(t + 2 < per_w)
            def _():
                for cc in in_copies(c + 2, b):
                    cc.start()
        return carry

    lax.fori_loop(0, per_w // 2, pair_body, 0)
    out_copy(base + per_w - 2, 0).wait()
    out_copy(base + per_w - 1, 1).wait()


def kernel(x, indices):
    b, ch, h, w = x.shape
    xf = x.reshape(-1)
    idxf = indices.astype(jnp.int32).reshape(-1)
    total_out = b * ch * 4 * h * w
    mesh = plsc.VectorSubcoreMesh(core_axis_name="c", subcore_axis_name="s")
    sc_call = functools.partial(
        pl.kernel,
        mesh=mesh,
        out_type=jax.ShapeDtypeStruct((total_out,), jnp.float32),
        scratch_types=[
            pltpu.VMEM((_CH_IN,), jnp.float32),
            pltpu.VMEM((_CH_IN,), jnp.float32),
            pltpu.VMEM((_CH_IN,), jnp.int32),
            pltpu.VMEM((_CH_IN,), jnp.int32),
            pltpu.VMEM((_CH_OUT,), jnp.float32),
            pltpu.VMEM((_CH_OUT,), jnp.float32),
            pltpu.SemaphoreType.DMA,
            pltpu.SemaphoreType.DMA,
            pltpu.SemaphoreType.DMA,
            pltpu.SemaphoreType.DMA,
            pltpu.SemaphoreType.DMA,
            pltpu.SemaphoreType.DMA,
        ],
        compiler_params=pltpu.CompilerParams(needs_layout_passes=False),
    )(_sc_body)
    out = sc_call(xf, idxf)
    return out.reshape(b, ch, 2 * h, 2 * w)


# speedup vs baseline: 1.6815x; 1.0001x over previous
import functools

import jax
import jax.numpy as jnp
from jax import lax
from jax.experimental import pallas as pl
from jax.experimental.pallas import tpu as pltpu
from jax.experimental.pallas import tpu_sc as plsc

_W = 192
_RI = 24
_CH_IN = _RI * _W
_CH_OUT = 4 * _CH_IN
_NW = 32
_NB = 4


def _sc_body(x_hbm, idx_hbm, out_hbm, *refs):
    xbs = refs[0:_NB]
    ibs = refs[_NB:2 * _NB]
    obs = refs[2 * _NB:3 * _NB]
    sxs = refs[3 * _NB:4 * _NB]
    sis = refs[4 * _NB:5 * _NB]
    sos = refs[5 * _NB:6 * _NB]
    n_chunks = x_hbm.shape[0] // _CH_IN
    per_w = n_chunks // _NW
    cpp = (_W * _W) // _CH_IN
    wid = lax.axis_index("s") * 2 + lax.axis_index("c")
    base = wid * per_w

    def in_copies(c, b):
        return (
            pltpu.make_async_copy(
                x_hbm.at[pl.ds(c * _CH_IN, _CH_IN)], xbs[b], sxs[b]),
            pltpu.make_async_copy(
                idx_hbm.at[pl.ds(c * _CH_IN, _CH_IN)], ibs[b], sis[b]),
        )

    def out_copy(c, b):
        return pltpu.make_async_copy(
            obs[b], out_hbm.at[pl.ds(c * _CH_OUT, _CH_OUT)], sos[b])

    for b in range(_NB):
        for cc in in_copies(base + b, b):
            cc.start()

    def ring_body(p, carry):
        for b in range(_NB):
            t = _NB * p + b
            c = base + t
            ob, xb, ib = obs[b], xbs[b], ibs[b]

            @pl.when(t >= _NB)
            def _():
                out_copy(c - _NB, b).wait()

            @plsc.parallel_loop(0, _CH_OUT, step=16, unroll=8)
            def _(z):
                ob[pl.ds(z, 16)] = jnp.zeros((16,), jnp.float32)

            for cc in in_copies(c, b):
                cc.wait()

            pbase = (c % cpp) * _CH_OUT

            @plsc.parallel_loop(0, _CH_IN, step=16, unroll=4)
            def _(k):
                xv = xb[pl.ds(k, 16)]
                iv = ib[pl.ds(k, 16)]
                plsc.store_scatter(ob, [iv - pbase], xv)

            out_copy(c, b).start()

            @pl.when(t + _NB < per_w)
            def _():
                for cc in in_copies(c + _NB, b):
                    cc.start()
        return carry

    lax.fori_loop(0, per_w // _NB, ring_body, 0)
    for b in range(_NB):
        out_copy(base + per_w - _NB + b, b).wait()


def kernel(x, indices):
    b, ch, h, w = x.shape
    xf = x.reshape(-1)
    idxf = indices.astype(jnp.int32).reshape(-1)
    total_out = b * ch * 4 * h * w
    mesh = plsc.VectorSubcoreMesh(core_axis_name="c", subcore_axis_name="s")
    scratch = (
        [pltpu.VMEM((_CH_IN,), jnp.float32) for _ in range(_NB)]
        + [pltpu.VMEM((_CH_IN,), jnp.int32) for _ in range(_NB)]
        + [pltpu.VMEM((_CH_OUT,), jnp.float32) for _ in range(_NB)]
        + [pltpu.SemaphoreType.DMA for _ in range(3 * _NB)]
    )
    sc_call = functools.partial(
        pl.kernel,
        mesh=mesh,
        out_type=jax.ShapeDtypeStruct((total_out,), jnp.float32),
        scratch_types=scratch,
        compiler_params=pltpu.CompilerParams(needs_layout_passes=False),
    )(_sc_body)
    out = sc_call(xf, idxf)
    return out.reshape(b, ch, 2 * h, 2 * w)
